# baseline (device time: 15696 ns/iter reference)
import jax
import jax.numpy as jnp
from jax import lax
from jax.experimental import pallas as pl
from jax.experimental.pallas import tpu as pltpu

N_Z = 4
N_G = 4


def kernel(Q, K, V):
    b, kv, h, d = K.shape
    bh = b * h
    scale = d ** -0.5
    pk = d + 2
    rg = bh // N_G

    Kt = K.transpose(0, 2, 3, 1).reshape(bh, d, kv).astype(jnp.bfloat16)
    Vt = V.transpose(0, 2, 3, 1).reshape(bh, d, kv).astype(jnp.bfloat16)
    Qt = Q.transpose(0, 2, 1, 3).reshape(bh, d).astype(jnp.bfloat16)

    def body(q_ref, k_ref, v_ref, o_ref, loc_ref, comm_ref,
             send_sems, recv_sems):
        my_x = lax.axis_index("x")
        my_y = lax.axis_index("y")
        my_z = lax.axis_index("z")

        barrier_sem = pltpu.get_barrier_semaphore()
        for j in range(1, N_Z):
            pl.semaphore_signal(
                barrier_sem, inc=1,
                device_id=(my_x, my_y, (my_z + j) % N_Z),
                device_id_type=pl.DeviceIdType.MESH,
            )

        sends = []
        for g in range(N_G):
            rs = slice(g * rg, (g + 1) * rg)
            qg = q_ref[rs]
            s = jnp.sum(k_ref[rs] * qg[:, :, None],
                        axis=1).astype(jnp.float32) * scale
            m_loc = jnp.max(s, axis=-1, keepdims=True)
            p = jnp.exp(s - m_loc)
            l_loc = jnp.sum(p, axis=-1, keepdims=True)
            o_loc = jnp.sum(v_ref[rs] * p.astype(jnp.bfloat16)[:, None, :],
                            axis=2).astype(jnp.float32)
            loc_ref[rs] = jnp.concatenate([o_loc, m_loc, l_loc], axis=1)

            if g == 0:
                pl.semaphore_wait(barrier_sem, N_Z - 1)

            for j in range(1, N_Z):
                rdma = pltpu.make_async_remote_copy(
                    src_ref=loc_ref.at[rs],
                    dst_ref=comm_ref.at[j - 1, rs],
                    send_sem=send_sems.at[j - 1, g],
                    recv_sem=recv_sems.at[j - 1, g],
                    device_id=(my_x, my_y, (my_z + j) % N_Z),
                    device_id_type=pl.DeviceIdType.MESH,
                )
                rdma.start()
                sends.append(rdma)

        for g in range(N_G):
            rs = slice(g * rg, (g + 1) * rg)
            for j in range(1, N_Z):
                pltpu.make_async_remote_copy(
                    src_ref=loc_ref.at[rs],
                    dst_ref=comm_ref.at[j - 1, rs],
                    send_sem=send_sems.at[j - 1, g],
                    recv_sem=recv_sems.at[j - 1, g],
                    device_id=(my_x, my_y, (my_z + j) % N_Z),
                    device_id_type=pl.DeviceIdType.MESH,
                ).wait_recv()

        mine = loc_ref[...]
        m_loc = mine[:, d:d + 1]
        m_max = m_loc
        for jj in range(N_Z - 1):
            m_max = jnp.maximum(m_max, comm_ref[jj, :, d:d + 1])
        sc = jnp.exp(m_loc - m_max)
        num = mine[:, 0:d] * sc
        den = mine[:, d + 1:d + 2] * sc
        for jj in range(N_Z - 1):
            sc = jnp.exp(comm_ref[jj, :, d:d + 1] - m_max)
            num = num + comm_ref[jj, :, 0:d] * sc
            den = den + comm_ref[jj, :, d + 1:d + 2] * sc
        o_ref[...] = num / den

        for rdma in sends:
            rdma.wait_send()

    out = pl.pallas_call(
        body,
        out_shape=jax.ShapeDtypeStruct((bh, d), jnp.float32),
        in_specs=[
            pl.BlockSpec(memory_space=pltpu.VMEM),
            pl.BlockSpec(memory_space=pltpu.VMEM),
            pl.BlockSpec(memory_space=pltpu.VMEM),
        ],
        out_specs=pl.BlockSpec(memory_space=pltpu.VMEM),
        scratch_shapes=[
            pltpu.VMEM((bh, pk), jnp.float32),
            pltpu.VMEM((N_Z - 1, bh, pk), jnp.float32),
            pltpu.SemaphoreType.DMA((N_Z - 1, N_G)),
            pltpu.SemaphoreType.DMA((N_Z - 1, N_G)),
        ],
        compiler_params=pltpu.CompilerParams(collective_id=0),
    )(Qt, Kt, Vt)

    return out.reshape(b, h, d)[:, None, :, :]
